# Initial kernel scaffold; baseline (speedup 1.0000x reference)
import jax
import jax.numpy as jnp
from jax.experimental import pallas as pl


def kernel(x, Wq, Wk, Wv):
    raise NotImplementedError("write your pallas kernel here")


# speedup vs baseline: 1.0153x; 1.0153x over previous
import math

import jax
import jax.numpy as jnp
from jax.experimental import pallas as pl
from jax.experimental.pallas import tpu as pltpu


def _scores_kernel(x_ref, wq_ref, wk_ref, o_ref):
    xv = x_ref[0]
    dn = (((1,), (1,)), ((), ()))
    q = jax.lax.dot_general(xv, wq_ref[0], dn,
                            preferred_element_type=jnp.float32)
    k = jax.lax.dot_general(xv, wk_ref[0], dn,
                            preferred_element_type=jnp.float32)
    o_ref[0] = jax.lax.dot_general(q.astype(jnp.bfloat16), k.astype(jnp.bfloat16),
                                   dn, preferred_element_type=jnp.float32)


def kernel(x, Wq, Wk, Wv):
    B, S, IN = x.shape
    HID = Wq.shape[0]
    D = 128
    H = HID // D
    scale = 1.0 / math.sqrt(D)

    xb = x.astype(jnp.bfloat16)
    wqb = (Wq * scale).reshape(H, D, IN).astype(jnp.bfloat16)
    wkb = Wk.reshape(H, D, IN).astype(jnp.bfloat16)

    scores = pl.pallas_call(
        _scores_kernel,
        out_shape=jax.ShapeDtypeStruct((B, S, H * S), jnp.float32),
        grid=(B, H),
        in_specs=[
            pl.BlockSpec((1, S, IN), lambda b, h: (b, 0, 0)),
            pl.BlockSpec((1, D, IN), lambda b, h: (h, 0, 0)),
            pl.BlockSpec((1, D, IN), lambda b, h: (h, 0, 0)),
        ],
        out_specs=pl.BlockSpec((1, S, S), lambda b, h: (b, 0, h)),
        compiler_params=pltpu.CompilerParams(
            dimension_semantics=("parallel", "arbitrary"),
            vmem_limit_bytes=56 * 1024 * 1024,
        ),
        name="qk_scores",
    )(xb, wqb, wkb)

    attn_weights = scores.reshape(B, S, H, S)
    attn_output = jnp.zeros((B, S, HID), dtype=x.dtype)
    return attn_output, attn_weights


# speedup vs baseline: 1.5573x; 1.5339x over previous
import math

import jax
import jax.numpy as jnp
from jax.experimental import pallas as pl
from jax.experimental.pallas import tpu as pltpu

_D = 128
_HG = 8
_BM = 512
_BQ = 128
_CH = 512


def _proj_kernel(x_ref, wq_ref, wk_ref, q_ref, k_ref):
    dn = (((1,), (1,)), ((), ()))
    xv = x_ref[...]
    q_ref[...] = jax.lax.dot_general(
        xv, wq_ref[...], dn, preferred_element_type=jnp.float32
    ).astype(jnp.bfloat16)
    k_ref[...] = jax.lax.dot_general(
        xv, wk_ref[...], dn, preferred_element_type=jnp.float32
    ).astype(jnp.bfloat16)


def _scores_kernel(mask_ref, q_ref, k_ref, o_ref):
    qv = q_ref[0]
    kv = k_ref[0]
    rep = jnp.repeat(qv, _HG, axis=0)
    lhs = rep * jnp.tile(mask_ref[...], (_BQ // 2, 1))
    out = jax.lax.dot_general(lhs, kv, (((1,), (1,)), ((), ())),
                              preferred_element_type=jnp.float32)
    o_ref[0] = out.reshape(_BQ, _HG, kv.shape[0])


def kernel(x, Wq, Wk, Wv):
    B, S, IN = x.shape
    HID = Wq.shape[0]
    H = HID // _D
    scale = 1.0 / math.sqrt(_D)

    xb = x.astype(jnp.bfloat16).reshape(B * S, IN)
    wqb = (Wq * scale).astype(jnp.bfloat16)
    wkb = Wk.astype(jnp.bfloat16)

    R = B * S
    q2, k2 = pl.pallas_call(
        _proj_kernel,
        out_shape=(
            jax.ShapeDtypeStruct((R, HID), jnp.bfloat16),
            jax.ShapeDtypeStruct((R, HID), jnp.bfloat16),
        ),
        grid=(R // _BM,),
        in_specs=[
            pl.BlockSpec((_BM, IN), lambda i: (i, 0)),
            pl.BlockSpec((HID, IN), lambda i: (0, 0)),
            pl.BlockSpec((HID, IN), lambda i: (0, 0)),
        ],
        out_specs=(
            pl.BlockSpec((_BM, HID), lambda i: (i, 0)),
            pl.BlockSpec((_BM, HID), lambda i: (i, 0)),
        ),
        compiler_params=pltpu.CompilerParams(
            dimension_semantics=("parallel",),
            vmem_limit_bytes=56 * 1024 * 1024,
        ),
        name="qk_proj",
    )(xb, wqb, wkb)

    qr = q2.reshape(B, S, HID)
    kr = k2.reshape(B, S, HID)

    mask16 = (jnp.arange(16, dtype=jnp.int32)[:, None] % _HG
              == jnp.arange(_HG * _D, dtype=jnp.int32)[None, :] // _D
              ).astype(jnp.bfloat16)

    attn_weights = pl.pallas_call(
        _scores_kernel,
        out_shape=jax.ShapeDtypeStruct((B, S, H, S), jnp.float32),
        grid=(B, H // _HG, S // _BQ),
        in_specs=[
            pl.BlockSpec((16, _HG * _D), lambda b, g, i: (0, 0)),
            pl.BlockSpec((1, _BQ, _HG * _D), lambda b, g, i: (b, i, g)),
            pl.BlockSpec((1, S, _HG * _D), lambda b, g, i: (b, 0, g)),
        ],
        out_specs=pl.BlockSpec((1, _BQ, _HG, S), lambda b, g, i: (b, i, g, 0)),
        compiler_params=pltpu.CompilerParams(
            dimension_semantics=("parallel", "arbitrary", "arbitrary"),
            vmem_limit_bytes=56 * 1024 * 1024,
        ),
        name="qk_scores",
    )(mask16, qr, kr)

    attn_output = jnp.zeros((B, S, HID), dtype=x.dtype)
    return attn_output, attn_weights


# speedup vs baseline: 1.6115x; 1.0348x over previous
import math

import jax
import jax.numpy as jnp
from jax.experimental import pallas as pl
from jax.experimental.pallas import tpu as pltpu

_D = 128
_HG = 8
_BM = 512
_BQ = 128
_CH = 512


def _proj_kernel(x_ref, wq_ref, wk_ref, q_ref, k_ref):
    dn = (((1,), (1,)), ((), ()))
    xv = x_ref[...].astype(jnp.bfloat16)
    q_ref[...] = jax.lax.dot_general(
        xv, wq_ref[...], dn, preferred_element_type=jnp.float32
    ).astype(jnp.bfloat16)
    k_ref[...] = jax.lax.dot_general(
        xv, wk_ref[...], dn, preferred_element_type=jnp.float32
    ).astype(jnp.bfloat16)


def _scores_kernel(mask_ref, q_ref, k_ref, o_ref):
    qv = q_ref[0]
    kv = k_ref[0]
    rep = jnp.repeat(qv, _HG, axis=0)
    lhs = rep * jnp.tile(mask_ref[...], (_BQ // 2, 1))
    out = jax.lax.dot_general(lhs, kv, (((1,), (1,)), ((), ())),
                              preferred_element_type=jnp.float32)
    o_ref[0] = out.reshape(_BQ, _HG, kv.shape[0])


def kernel(x, Wq, Wk, Wv):
    B, S, IN = x.shape
    HID = Wq.shape[0]
    H = HID // _D
    scale = 1.0 / math.sqrt(_D)

    xb = x.reshape(B * S, IN)
    wqb = (Wq * scale).astype(jnp.bfloat16)
    wkb = Wk.astype(jnp.bfloat16)

    R = B * S
    q2, k2 = pl.pallas_call(
        _proj_kernel,
        out_shape=(
            jax.ShapeDtypeStruct((R, HID), jnp.bfloat16),
            jax.ShapeDtypeStruct((R, HID), jnp.bfloat16),
        ),
        grid=(R // _BM,),
        in_specs=[
            pl.BlockSpec((_BM, IN), lambda i: (i, 0)),
            pl.BlockSpec((HID, IN), lambda i: (0, 0)),
            pl.BlockSpec((HID, IN), lambda i: (0, 0)),
        ],
        out_specs=(
            pl.BlockSpec((_BM, HID), lambda i: (i, 0)),
            pl.BlockSpec((_BM, HID), lambda i: (i, 0)),
        ),
        compiler_params=pltpu.CompilerParams(
            dimension_semantics=("parallel",),
            vmem_limit_bytes=56 * 1024 * 1024,
        ),
        name="qk_proj",
    )(xb, wqb, wkb)

    qr = q2.reshape(B, S, HID)
    kr = k2.reshape(B, S, HID)

    mask16 = (jnp.arange(16, dtype=jnp.int32)[:, None] % _HG
              == jnp.arange(_HG * _D, dtype=jnp.int32)[None, :] // _D
              ).astype(jnp.bfloat16)

    attn_weights = pl.pallas_call(
        _scores_kernel,
        out_shape=jax.ShapeDtypeStruct((B, S, H, S), jnp.float32),
        grid=(B, H // _HG, S // _BQ),
        in_specs=[
            pl.BlockSpec((16, _HG * _D), lambda b, g, i: (0, 0)),
            pl.BlockSpec((1, _BQ, _HG * _D), lambda b, g, i: (b, i, g)),
            pl.BlockSpec((1, S, _HG * _D), lambda b, g, i: (b, 0, g)),
        ],
        out_specs=pl.BlockSpec((1, _BQ, _HG, S), lambda b, g, i: (b, i, g, 0)),
        compiler_params=pltpu.CompilerParams(
            dimension_semantics=("parallel", "arbitrary", "arbitrary"),
            vmem_limit_bytes=56 * 1024 * 1024,
        ),
        name="qk_scores",
    )(mask16, qr, kr)

    attn_output = jnp.zeros((B, S, HID), dtype=x.dtype)
    return attn_output, attn_weights


# speedup vs baseline: 1.6646x; 1.0329x over previous
import math

import jax
import jax.numpy as jnp
from jax.experimental import pallas as pl
from jax.experimental.pallas import tpu as pltpu

_D = 128
_HG = 8
_BM = 512
_BQ = 256
_CH = 1024


def _proj_kernel(x_ref, wq_ref, wk_ref, q_ref, k_ref, z_ref):
    dn = (((1,), (1,)), ((), ()))
    xv = x_ref[...].astype(jnp.bfloat16)
    q_ref[...] = jax.lax.dot_general(
        xv, wq_ref[...], dn, preferred_element_type=jnp.float32
    ).astype(jnp.bfloat16)
    k_ref[...] = jax.lax.dot_general(
        xv, wk_ref[...], dn, preferred_element_type=jnp.float32
    ).astype(jnp.bfloat16)
    z_ref[...] = jnp.zeros_like(z_ref)


def _scores_kernel(mask_ref, q_ref, k_ref, o_ref):
    qv = q_ref[0]
    kv = k_ref[0]
    rep = jnp.repeat(qv, _HG, axis=0)
    lhs = rep * jnp.tile(mask_ref[...], (_BQ // 2, 1))
    S = kv.shape[0]
    for c in range(S // _CH):
        out = jax.lax.dot_general(lhs, kv[c * _CH:(c + 1) * _CH, :],
                                  (((1,), (1,)), ((), ())),
                                  preferred_element_type=jnp.float32)
        o_ref[0, :, :, c * _CH:(c + 1) * _CH] = out.reshape(_BQ, _HG, _CH)


def kernel(x, Wq, Wk, Wv):
    B, S, IN = x.shape
    HID = Wq.shape[0]
    H = HID // _D
    scale = 1.0 / math.sqrt(_D)

    xb = x.reshape(B * S, IN)
    wqb = (Wq * scale).astype(jnp.bfloat16)
    wkb = Wk.astype(jnp.bfloat16)

    R = B * S
    q2, k2, zeros = pl.pallas_call(
        _proj_kernel,
        out_shape=(
            jax.ShapeDtypeStruct((R, HID), jnp.bfloat16),
            jax.ShapeDtypeStruct((R, HID), jnp.bfloat16),
            jax.ShapeDtypeStruct((R, HID), jnp.float32),
        ),
        grid=(R // _BM,),
        in_specs=[
            pl.BlockSpec((_BM, IN), lambda i: (i, 0)),
            pl.BlockSpec((HID, IN), lambda i: (0, 0)),
            pl.BlockSpec((HID, IN), lambda i: (0, 0)),
        ],
        out_specs=(
            pl.BlockSpec((_BM, HID), lambda i: (i, 0)),
            pl.BlockSpec((_BM, HID), lambda i: (i, 0)),
            pl.BlockSpec((_BM, HID), lambda i: (i, 0)),
        ),
        compiler_params=pltpu.CompilerParams(
            dimension_semantics=("parallel",),
            vmem_limit_bytes=56 * 1024 * 1024,
        ),
        name="qk_proj",
    )(xb, wqb, wkb)

    qr = q2.reshape(B, S, HID)
    kr = k2.reshape(B, S, HID)

    mask16 = (jnp.arange(16, dtype=jnp.int32)[:, None] % _HG
              == jnp.arange(_HG * _D, dtype=jnp.int32)[None, :] // _D
              ).astype(jnp.bfloat16)

    attn_weights = pl.pallas_call(
        _scores_kernel,
        out_shape=jax.ShapeDtypeStruct((B, S, H, S), jnp.float32),
        grid=(B, H // _HG, S // _BQ),
        in_specs=[
            pl.BlockSpec((16, _HG * _D), lambda b, g, i: (0, 0)),
            pl.BlockSpec((1, _BQ, _HG * _D), lambda b, g, i: (b, i, g)),
            pl.BlockSpec((1, S, _HG * _D), lambda b, g, i: (b, 0, g)),
        ],
        out_specs=pl.BlockSpec((1, _BQ, _HG, S), lambda b, g, i: (b, i, g, 0)),
        compiler_params=pltpu.CompilerParams(
            dimension_semantics=("parallel", "arbitrary", "arbitrary"),
            vmem_limit_bytes=56 * 1024 * 1024,
        ),
        name="qk_scores",
    )(mask16, qr, kr)

    attn_output = zeros.reshape(B, S, HID)
    return attn_output, attn_weights
